# baseline (speedup 1.0000x reference)
import functools

import jax
import jax.numpy as jnp
from jax import lax
from jax.experimental import pallas as pl
from jax.experimental.pallas import tpu as pltpu
from jax.experimental.pallas import tpu_sc as plsc

N = 50000
E = 800000
HID = 64
OUT = 32
D1 = 16
D2 = OUT // 2

NC = 2
NS = 16
L = 16

NPAD = 51200
SPT = NPAD // NS
CH = 128
NJ = 408
EPAD = NS * NJ * CH
NJD = NJ // NC
G = 68
R = 6

BLK = 1024


def _deg_body(col3, ew3, out, acc, colst, ewst, zb, cob):
    c = lax.axis_index("c")
    s = lax.axis_index("s")
    for k in range(CH // L):
        zb[pl.ds(k * L, L)] = jnp.zeros((L,), jnp.float32)

    def zloop(t, carry):
        pltpu.sync_copy(zb, acc.at[pl.ds(s * SPT + t * CH, CH)])
        return carry

    lax.fori_loop(0, SPT // CH, zloop, 0)
    plsc.subcore_barrier()

    j0 = c * NJD

    def gloop(g, carry):
        base = j0 + g * G
        pltpu.sync_copy(col3.at[s, pl.ds(base, G)], colst)
        pltpu.sync_copy(ew3.at[s, pl.ds(base, G)], ewst)

        def cloop(jj, carry2):
            pltpu.sync_copy(ewst.at[jj], acc.at[colst.at[jj]], add=True)
            return carry2

        lax.fori_loop(0, G, cloop, 0)
        return carry

    lax.fori_loop(0, NJD // G, gloop, 0)
    plsc.subcore_barrier()

    def oloop(t, carry):
        off = s * SPT + t * CH
        pltpu.sync_copy(acc.at[pl.ds(off, CH)], cob)
        pltpu.sync_copy(cob, out.at[pl.ds(c * NPAD + off, CH)])
        return carry

    lax.fori_loop(0, SPT // CH, oloop, 0)


def _make_deg_call():
    mesh = plsc.VectorSubcoreMesh(
        core_axis_name="c", subcore_axis_name="s", num_cores=NC, num_subcores=NS)
    return pl.kernel(
        _deg_body,
        out_type=jax.ShapeDtypeStruct((NC * NPAD,), jnp.float32),
        mesh=mesh,
        compiler_params=pltpu.CompilerParams(use_tc_tiling_on_sc=False),
        scratch_types=[
            pltpu.VMEM_SHARED((NPAD,), jnp.float32),
            pltpu.VMEM((G, CH), jnp.int32),
            pltpu.VMEM((G, CH), jnp.float32),
            pltpu.VMEM((CH,), jnp.float32),
            pltpu.VMEM((CH,), jnp.float32),
        ],
    )


def _edge_pass_body(Dh, split_edges, edata, ew3, tbl, out,
                    acc, echt, ewch, *rest):
    c = lax.axis_index("c")
    s = lax.axis_index("s")
    if split_edges:
        idxb = None
        rows = rest[0]
        sems = rest[1:]
    else:
        idxb, rows = rest[0], rest[1]
        sems = rest[2:]
    stsems = sems[:R]
    esems = sems[R:2 * R]
    gsems = sems[2 * R:3 * R]
    ssems = sems[3 * R:]
    if split_edges:
        njloc = NJD
        j0 = c * NJD
        shift = None
    else:
        njloc = NJ
        j0 = 0
        shift = c * NPAD

    def zrow(r, carry):
        for k in range(Dh // L):
            rows[0, r, pl.ds(k * L, L)] = jnp.zeros((L,), jnp.float32)
        return carry

    lax.fori_loop(0, CH, zrow, 0, unroll=8)

    def zloop(t, carry):
        pltpu.async_copy(rows.at[0], acc.at[pl.ds(s * SPT + t * CH, CH)],
                         gsems[0])
        return carry

    lax.fori_loop(0, SPT // CH, zloop, 0)

    def zdrain(t, carry):
        pltpu.make_async_copy(
            rows.at[0], acc.at[pl.ds(s * SPT, CH)], gsems[0]).wait()
        return carry

    lax.fori_loop(0, SPT // CH, zdrain, 0)
    plsc.subcore_barrier()

    def start_stage(q, jj):
        pltpu.async_copy(edata.at[s, j0 + jj], echt.at[q], stsems[q])
        pltpu.async_copy(ew3.at[s, j0 + jj], ewch.at[q], esems[q])

    def wait_stage(q):
        pltpu.make_async_copy(edata.at[s, 0], echt.at[q], stsems[q]).wait()
        pltpu.make_async_copy(ew3.at[s, 0], ewch.at[q], esems[q]).wait()

    def gather_idx(q):
        if shift is None:
            return echt.at[q, 0]
        for k in range(CH // L):
            idxb[q, pl.ds(k * L, L)] = echt[q, 0, pl.ds(k * L, L)] + shift
        return idxb.at[q]

    def start_gather(q):
        pltpu.async_copy(tbl.at[gather_idx(q)], rows.at[q], gsems[q])

    def wait_gather(q):
        iref = echt.at[q, 0] if shift is None else idxb.at[q]
        pltpu.make_async_copy(tbl.at[iref], rows.at[q], gsems[q]).wait()

    def start_scatter(q):
        pltpu.async_copy(rows.at[q], acc.at[echt.at[q, 1]], ssems[q], add=True)

    def wait_scatter(q):
        pltpu.make_async_copy(rows.at[q], acc.at[echt.at[q, 1]], ssems[q]).wait()

    def scale_rows(q):
        def eloop(i16, carry):
            colo = pl.multiple_of(i16 * L, L)
            ew16 = ewch[q, pl.ds(colo, L)]
            for u in range(L):
                ewb = ew16.at[jnp.full((L,), u, jnp.int32)].get(
                    mode="promise_in_bounds")
                i = i16 * L + u
                for k in range(Dh // L):
                    rows[q, i, pl.ds(k * L, L)] = (
                        rows[q, i, pl.ds(k * L, L)] * ewb)
            return carry

        lax.fori_loop(0, CH // L, eloop, 0)

    for q in range(3):
        start_stage(q, q)
    for q in range(2):
        wait_stage(q)
        start_gather(q)

    def six(j6, carry):
        for q in range(R):
            jj = j6 * R + q
            wait_gather(q)
            scale_rows(q)
            start_scatter(q)

            qs = (q + 3) % R

            @pl.when(jj + 3 < njloc)
            def _stage():
                @pl.when(jj >= 3)
                def _drain():
                    wait_scatter(qs)

                start_stage(qs, jj + 3)

            qg = (q + 2) % R

            @pl.when(jj + 2 < njloc)
            def _gather():
                wait_stage(qg)
                start_gather(qg)
        return carry

    lax.fori_loop(0, njloc // R, six, 0)
    for jj in range(njloc - R, njloc):
        wait_scatter(jj % R)

    plsc.subcore_barrier()

    def oloop(t, carry):
        off = s * SPT + t * CH
        pltpu.async_copy(acc.at[pl.ds(off, CH)], out.at[c, pl.ds(off, CH)],
                         gsems[1])
        return carry

    lax.fori_loop(0, SPT // CH, oloop, 0)

    def odrain(t, carry):
        pltpu.make_async_copy(
            acc.at[pl.ds(s * SPT, CH)], out.at[c, pl.ds(s * SPT, CH)],
            gsems[1]).wait()
        return carry

    lax.fori_loop(0, SPT // CH, odrain, 0)


def _make_edge_pass(Dh, split_edges):
    mesh = plsc.VectorSubcoreMesh(
        core_axis_name="c", subcore_axis_name="s", num_cores=NC, num_subcores=NS)
    return pl.kernel(
        functools.partial(_edge_pass_body, Dh, split_edges),
        out_type=jax.ShapeDtypeStruct((NC, NPAD, Dh), jnp.float32),
        mesh=mesh,
        compiler_params=pltpu.CompilerParams(use_tc_tiling_on_sc=False),
        scratch_types=[
            pltpu.VMEM_SHARED((NPAD, Dh), jnp.float32),
            pltpu.VMEM((R, 2, CH), jnp.int32),
            pltpu.VMEM((R, CH), jnp.float32),
        ] + ([] if split_edges else [pltpu.VMEM((R, CH), jnp.int32)])
        + [pltpu.VMEM((R, CH, Dh), jnp.float32)]
        + [pltpu.SemaphoreType.DMA] * (4 * R),
    )


def _pre_body(degs_ref, x_ref, w1_ref, deg_ref, dis_ref, xw_ref, x2p_ref):
    deg = degs_ref[0] + degs_ref[1] + 1.0
    dis = lax.rsqrt(deg)
    x = x_ref[...]
    xw = jnp.dot(x, w1_ref[...], preferred_element_type=jnp.float32)
    deg_ref[...] = deg
    dis_ref[...] = dis
    xw_ref[...] = xw
    x2p_ref[...] = jnp.concatenate(
        [dis * x, jnp.zeros((x.shape[0], D1 - 2), jnp.float32)], axis=1)


def _mid_body(accp_ref, xw_ref, deg_ref, dis_ref, w1_ref, b1_ref, w2_ref,
              hw_ref, hs_ref):
    deg = deg_ref[...]
    dis = dis_ref[...]
    a2 = accp_ref[0, :, 0:2] + accp_ref[1, :, 0:2]
    acc1 = jnp.dot(a2, w1_ref[...], preferred_element_type=jnp.float32)
    h = jax.nn.relu(dis * acc1 + xw_ref[...] / deg + b1_ref[...])
    hw = jnp.dot(h, w2_ref[...], preferred_element_type=jnp.float32)
    hw_ref[...] = hw
    hs_ref[...] = dis * hw


def _post_body(acc2_ref, hw_ref, deg_ref, dis_ref, b2_ref, out_ref):
    deg = deg_ref[...]
    dis = dis_ref[...]
    acc = acc2_ref[0] + acc2_ref[1]
    out_ref[...] = jax.nn.relu(dis * acc + hw_ref[...] / deg + b2_ref[...])


def _row_spec(cols, blk=BLK):
    return pl.BlockSpec((blk, cols), lambda i: (i, 0))


def _stk_spec(cols, blk=BLK):
    return pl.BlockSpec((NC, blk, cols), lambda i: (0, i, 0))


def _full_spec(shape):
    return pl.BlockSpec(shape, lambda i: tuple(0 for _ in shape))


def _tc_pre(degs, x, w1):
    return pl.pallas_call(
        _pre_body,
        grid=(NPAD // BLK,),
        in_specs=[_stk_spec(1), _row_spec(2), _full_spec((2, HID))],
        out_specs=[_row_spec(1), _row_spec(1), _row_spec(HID), _row_spec(D1)],
        out_shape=[
            jax.ShapeDtypeStruct((NPAD, 1), jnp.float32),
            jax.ShapeDtypeStruct((NPAD, 1), jnp.float32),
            jax.ShapeDtypeStruct((NPAD, HID), jnp.float32),
            jax.ShapeDtypeStruct((NPAD, D1), jnp.float32),
        ],
    )(degs, x, w1)


def _tc_mid(accp, xw, deg, dis, w1, b1, w2):
    return pl.pallas_call(
        _mid_body,
        grid=(NPAD // BLK,),
        in_specs=[_stk_spec(D1), _row_spec(HID), _row_spec(1), _row_spec(1),
                  _full_spec((2, HID)), _full_spec((1, HID)),
                  _full_spec((HID, OUT))],
        out_specs=[_row_spec(OUT), _row_spec(OUT)],
        out_shape=[
            jax.ShapeDtypeStruct((NPAD, OUT), jnp.float32),
            jax.ShapeDtypeStruct((NPAD, OUT), jnp.float32),
        ],
    )(accp, xw, deg, dis, w1, b1, w2)


def _tc_post(acc2, hw, deg, dis, b2):
    nblk = 1000
    return pl.pallas_call(
        _post_body,
        grid=(N // nblk,),
        in_specs=[_stk_spec(OUT, nblk), _row_spec(OUT, nblk),
                  _row_spec(1, nblk), _row_spec(1, nblk),
                  _full_spec((1, OUT))],
        out_specs=_row_spec(OUT, nblk),
        out_shape=jax.ShapeDtypeStruct((N, OUT), jnp.float32),
    )(acc2, hw, deg, dis, b2)


def kernel(coords, edge_index, edge_weight, node_masks, W1, b1, W2, b2):
    x = coords[0]
    row = edge_index[0].astype(jnp.int32)
    col = edge_index[1].astype(jnp.int32)
    ew = edge_weight

    pad = EPAD - E
    row3 = jnp.pad(row, (0, pad)).reshape(NS, NJ, CH)
    col3 = jnp.pad(col, (0, pad), constant_values=N).reshape(NS, NJ, CH)
    ew3 = jnp.pad(ew, (0, pad)).reshape(NS, NJ, CH)
    edata = jnp.stack([row3, col3], axis=2)
    xp = jnp.pad(x, ((0, NPAD - N), (0, 0)))

    degs = _make_deg_call()(col3, ew3)

    deg, dis, xw, x2p = _tc_pre(degs.reshape(NC, NPAD, 1), xp, W1)

    acc1p = _make_edge_pass(D1, True)(edata, ew3, x2p)
    hw, hs = _tc_mid(acc1p, xw, deg, dis, W1, b1[None, :], W2)

    acc2p = _make_edge_pass(OUT, True)(edata, ew3, hs)
    out = _tc_post(acc2p, hw, deg, dis, b2[None, :])
    return out[None]


# speedup vs baseline: 1.0657x; 1.0657x over previous
import functools

import jax
import jax.numpy as jnp
from jax import lax
from jax.experimental import pallas as pl
from jax.experimental.pallas import tpu as pltpu
from jax.experimental.pallas import tpu_sc as plsc

N = 50000
E = 800000
HID = 64
OUT = 32
D1 = 16
D2 = OUT // 2

NC = 2
NS = 16
L = 16

NPAD = 51200
SPT = NPAD // NS
CH = 128
NJ = 408
EPAD = NS * NJ * CH
NJD = NJ // NC
G = 68
R = 6

BLK = 1024


def _deg_body(col3, ew3, out, acc, colst, ewst, zb, cob):
    c = lax.axis_index("c")
    s = lax.axis_index("s")
    for k in range(CH // L):
        zb[pl.ds(k * L, L)] = jnp.zeros((L,), jnp.float32)

    def zloop(t, carry):
        pltpu.sync_copy(zb, acc.at[pl.ds(s * SPT + t * CH, CH)])
        return carry

    lax.fori_loop(0, SPT // CH, zloop, 0)
    plsc.subcore_barrier()

    j0 = c * NJD

    def gloop(g, carry):
        base = j0 + g * G
        pltpu.sync_copy(col3.at[s, pl.ds(base, G)], colst)
        pltpu.sync_copy(ew3.at[s, pl.ds(base, G)], ewst)

        def cloop(jj, carry2):
            pltpu.sync_copy(ewst.at[jj], acc.at[colst.at[jj]], add=True)
            return carry2

        lax.fori_loop(0, G, cloop, 0)
        return carry

    lax.fori_loop(0, NJD // G, gloop, 0)
    plsc.subcore_barrier()

    def oloop(t, carry):
        off = s * SPT + t * CH
        pltpu.sync_copy(acc.at[pl.ds(off, CH)], cob)
        pltpu.sync_copy(cob, out.at[pl.ds(c * NPAD + off, CH)])
        return carry

    lax.fori_loop(0, SPT // CH, oloop, 0)


def _make_deg_call():
    mesh = plsc.VectorSubcoreMesh(
        core_axis_name="c", subcore_axis_name="s", num_cores=NC, num_subcores=NS)
    return pl.kernel(
        _deg_body,
        out_type=jax.ShapeDtypeStruct((NC * NPAD,), jnp.float32),
        mesh=mesh,
        compiler_params=pltpu.CompilerParams(use_tc_tiling_on_sc=False),
        scratch_types=[
            pltpu.VMEM_SHARED((NPAD,), jnp.float32),
            pltpu.VMEM((G, CH), jnp.int32),
            pltpu.VMEM((G, CH), jnp.float32),
            pltpu.VMEM((CH,), jnp.float32),
            pltpu.VMEM((CH,), jnp.float32),
        ],
    )


def _edge_pass_body(Dh, split_edges, edata, ew3, tbl, out,
                    acc, echt, ewch, *rest):
    c = lax.axis_index("c")
    s = lax.axis_index("s")
    if split_edges:
        idxb = None
        rows = rest[0]
        sems = rest[1:]
    else:
        idxb, rows = rest[0], rest[1]
        sems = rest[2:]
    stsems = sems[:R]
    esems = sems[R:2 * R]
    gsems = sems[2 * R:3 * R]
    ssems = sems[3 * R:]
    if split_edges:
        njloc = NJD
        j0 = c * NJD
        shift = None
    else:
        njloc = NJ
        j0 = 0
        shift = c * NPAD

    def zrow(r, carry):
        for k in range(Dh // L):
            rows[0, r, pl.ds(k * L, L)] = jnp.zeros((L,), jnp.float32)
        return carry

    lax.fori_loop(0, CH, zrow, 0, unroll=8)

    def zloop(t, carry):
        pltpu.async_copy(rows.at[0], acc.at[pl.ds(s * SPT + t * CH, CH)],
                         gsems[0])
        return carry

    lax.fori_loop(0, SPT // CH, zloop, 0)

    def zdrain(t, carry):
        pltpu.make_async_copy(
            rows.at[0], acc.at[pl.ds(s * SPT, CH)], gsems[0]).wait()
        return carry

    lax.fori_loop(0, SPT // CH, zdrain, 0)
    plsc.subcore_barrier()

    def start_stage(q, jj):
        pltpu.async_copy(edata.at[s, j0 + jj], echt.at[q], stsems[q])
        pltpu.async_copy(ew3.at[s, j0 + jj], ewch.at[q], esems[q])

    def wait_stage(q):
        pltpu.make_async_copy(edata.at[s, 0], echt.at[q], stsems[q]).wait()
        pltpu.make_async_copy(ew3.at[s, 0], ewch.at[q], esems[q]).wait()

    def gather_idx(q):
        if shift is None:
            return echt.at[q, 0]
        for k in range(CH // L):
            idxb[q, pl.ds(k * L, L)] = echt[q, 0, pl.ds(k * L, L)] + shift
        return idxb.at[q]

    def start_gather(q):
        pltpu.async_copy(tbl.at[gather_idx(q)], rows.at[q], gsems[q])

    def wait_gather(q):
        iref = echt.at[q, 0] if shift is None else idxb.at[q]
        pltpu.make_async_copy(tbl.at[iref], rows.at[q], gsems[q]).wait()

    def start_scatter(q):
        pltpu.async_copy(rows.at[q], acc.at[echt.at[q, 1]], ssems[q], add=True)

    def wait_scatter(q):
        pltpu.make_async_copy(rows.at[q], acc.at[echt.at[q, 1]], ssems[q]).wait()

    def scale_rows(q):
        def eloop(i16, carry):
            colo = pl.multiple_of(i16 * L, L)
            ew16 = ewch[q, pl.ds(colo, L)]
            for u in range(L):
                ewb = ew16.at[jnp.full((L,), u, jnp.int32)].get(
                    mode="promise_in_bounds")
                i = i16 * L + u
                for k in range(Dh // L):
                    rows[q, i, pl.ds(k * L, L)] = (
                        rows[q, i, pl.ds(k * L, L)] * ewb)
            return carry

        lax.fori_loop(0, CH // L, eloop, 0)

    for q in range(3):
        start_stage(q, q)
    for q in range(2):
        wait_stage(q)
        start_gather(q)

    def six(j6, carry):
        for q in range(R):
            jj = j6 * R + q
            wait_gather(q)
            scale_rows(q)
            start_scatter(q)

            qs = (q + 3) % R

            @pl.when(jj + 3 < njloc)
            def _stage():
                @pl.when(jj >= 3)
                def _drain():
                    wait_scatter(qs)

                start_stage(qs, jj + 3)

            qg = (q + 2) % R

            @pl.when(jj + 2 < njloc)
            def _gather():
                wait_stage(qg)
                start_gather(qg)
        return carry

    lax.fori_loop(0, njloc // R, six, 0)
    for jj in range(njloc - R, njloc):
        wait_scatter(jj % R)

    plsc.subcore_barrier()

    def oloop(t, carry):
        off = s * SPT + t * CH
        pltpu.async_copy(acc.at[pl.ds(off, CH)], out.at[c, pl.ds(off, CH)],
                         gsems[1])
        return carry

    lax.fori_loop(0, SPT // CH, oloop, 0)

    def odrain(t, carry):
        pltpu.make_async_copy(
            acc.at[pl.ds(s * SPT, CH)], out.at[c, pl.ds(s * SPT, CH)],
            gsems[1]).wait()
        return carry

    lax.fori_loop(0, SPT // CH, odrain, 0)


def _make_edge_pass(Dh, split_edges):
    mesh = plsc.VectorSubcoreMesh(
        core_axis_name="c", subcore_axis_name="s", num_cores=NC, num_subcores=NS)
    return pl.kernel(
        functools.partial(_edge_pass_body, Dh, split_edges),
        out_type=jax.ShapeDtypeStruct((NC, NPAD, Dh), jnp.float32),
        mesh=mesh,
        compiler_params=pltpu.CompilerParams(use_tc_tiling_on_sc=False),
        scratch_types=[
            pltpu.VMEM_SHARED((NPAD, Dh), jnp.float32),
            pltpu.VMEM((R, 2, CH), jnp.int32),
            pltpu.VMEM((R, CH), jnp.float32),
        ] + ([] if split_edges else [pltpu.VMEM((R, CH), jnp.int32)])
        + [pltpu.VMEM((R, CH, Dh), jnp.float32)]
        + [pltpu.SemaphoreType.DMA] * (4 * R),
    )


def _pre_body(degs_ref, x_ref, w1_ref, deg_ref, dis_ref, xw_ref, x2p_ref):
    deg = degs_ref[0] + degs_ref[1] + 1.0
    dis = lax.rsqrt(deg)
    x = x_ref[...]
    xw = jnp.dot(x, w1_ref[...], preferred_element_type=jnp.float32)
    deg_ref[...] = deg
    dis_ref[...] = dis
    xw_ref[...] = xw
    x2p_ref[...] = jnp.concatenate(
        [dis * x, jnp.zeros((x.shape[0], D1 - 2), jnp.float32)], axis=1)


def _mid_body(accp_ref, xw_ref, deg_ref, dis_ref, w1_ref, b1_ref, w2_ref,
              hw_ref, hs2_ref):
    deg = deg_ref[...]
    dis = dis_ref[...]
    a2 = accp_ref[0, :, 0:2] + accp_ref[1, :, 0:2]
    acc1 = jnp.dot(a2, w1_ref[...], preferred_element_type=jnp.float32)
    h = jax.nn.relu(dis * acc1 + xw_ref[...] / deg + b1_ref[...])
    hw = jnp.dot(h, w2_ref[...], preferred_element_type=jnp.float32)
    hs = dis * hw
    hw_ref[...] = hw
    hs2_ref[0] = hs[:, :OUT // 2]
    hs2_ref[1] = hs[:, OUT // 2:]


def _post_body(acc2_ref, hw_ref, deg_ref, dis_ref, b2_ref, out_ref):
    deg = deg_ref[...]
    dis = dis_ref[...]
    acc = jnp.concatenate([acc2_ref[0], acc2_ref[1]], axis=1)
    out_ref[...] = jax.nn.relu(dis * acc + hw_ref[...] / deg + b2_ref[...])


def _row_spec(cols, blk=BLK):
    return pl.BlockSpec((blk, cols), lambda i: (i, 0))


def _stk_spec(cols, blk=BLK):
    return pl.BlockSpec((NC, blk, cols), lambda i: (0, i, 0))


def _full_spec(shape):
    return pl.BlockSpec(shape, lambda i: tuple(0 for _ in shape))


def _tc_pre(degs, x, w1):
    return pl.pallas_call(
        _pre_body,
        grid=(NPAD // BLK,),
        in_specs=[_stk_spec(1), _row_spec(2), _full_spec((2, HID))],
        out_specs=[_row_spec(1), _row_spec(1), _row_spec(HID), _row_spec(D1)],
        out_shape=[
            jax.ShapeDtypeStruct((NPAD, 1), jnp.float32),
            jax.ShapeDtypeStruct((NPAD, 1), jnp.float32),
            jax.ShapeDtypeStruct((NPAD, HID), jnp.float32),
            jax.ShapeDtypeStruct((NPAD, D1), jnp.float32),
        ],
    )(degs, x, w1)


def _tc_mid(accp, xw, deg, dis, w1, b1, w2):
    return pl.pallas_call(
        _mid_body,
        grid=(NPAD // BLK,),
        in_specs=[_stk_spec(D1), _row_spec(HID), _row_spec(1), _row_spec(1),
                  _full_spec((2, HID)), _full_spec((1, HID)),
                  _full_spec((HID, OUT))],
        out_specs=[_row_spec(OUT), _stk_spec(OUT // 2)],
        out_shape=[
            jax.ShapeDtypeStruct((NPAD, OUT), jnp.float32),
            jax.ShapeDtypeStruct((NC, NPAD, OUT // 2), jnp.float32),
        ],
    )(accp, xw, deg, dis, w1, b1, w2)


def _tc_post(acc2, hw, deg, dis, b2):
    nblk = 1000
    return pl.pallas_call(
        _post_body,
        grid=(N // nblk,),
        in_specs=[_stk_spec(OUT // 2, nblk), _row_spec(OUT, nblk),
                  _row_spec(1, nblk), _row_spec(1, nblk),
                  _full_spec((1, OUT))],
        out_specs=_row_spec(OUT, nblk),
        out_shape=jax.ShapeDtypeStruct((N, OUT), jnp.float32),
    )(acc2, hw, deg, dis, b2)


def kernel(coords, edge_index, edge_weight, node_masks, W1, b1, W2, b2):
    x = coords[0]
    row = edge_index[0].astype(jnp.int32)
    col = edge_index[1].astype(jnp.int32)
    ew = edge_weight

    pad = EPAD - E
    row3 = jnp.pad(row, (0, pad)).reshape(NS, NJ, CH)
    col3 = jnp.pad(col, (0, pad), constant_values=N).reshape(NS, NJ, CH)
    ew3 = jnp.pad(ew, (0, pad)).reshape(NS, NJ, CH)
    edata = jnp.stack([row3, col3], axis=2)
    xp = jnp.pad(x, ((0, NPAD - N), (0, 0)))

    degs = _make_deg_call()(col3, ew3)

    deg, dis, xw, x2p = _tc_pre(degs.reshape(NC, NPAD, 1), xp, W1)

    acc1p = _make_edge_pass(D1, True)(edata, ew3, x2p)
    hw, hs2 = _tc_mid(acc1p, xw, deg, dis, W1, b1[None, :], W2)

    acc2 = _make_edge_pass(D2, False)(
        edata, ew3, hs2.reshape(NC * NPAD, OUT // 2))
    out = _tc_post(acc2, hw, deg, dis, b2[None, :])
    return out[None]
